# Initial kernel scaffold; baseline (speedup 1.0000x reference)
import jax
import jax.numpy as jnp
from jax.experimental import pallas as pl


def kernel(wd_x, wd_y, in_emb, out_emb, in_bias, out_bias, co_matrix):
    raise NotImplementedError("write your pallas kernel here")


# speedup vs baseline: 568.9030x; 568.9030x over previous
import functools

import jax
import jax.numpy as jnp
from jax import lax
from jax.experimental import pallas as pl
from jax.experimental.pallas import tpu as pltpu
from jax.experimental.pallas import tpu_sc as plsc

VOCAB = 1000
DIM = 64
B = 16384
L = 200
VPAD = 1024
LPAD = 208
ROWB = 128

NC = 2
NS = 16
NW = NC * NS
BPW = B // NW
CH = 16
NCHUNK = BPW // CH
NV = LPAD // 16
NACC = 8


def _q_body(a_ref, bt_ref, bx_ref, by_ref, c_ref, q_ref):
    m = lax.dot_general(a_ref[...], bt_ref[...], (((1,), (0,)), ((), ())),
                        preferred_element_type=jnp.float32)
    c = c_ref[...]
    w = jnp.where(c < 100.0, jnp.exp(0.75 * jnp.log(c * 0.01)), 1.0)
    d = m + bx_ref[...] + by_ref[...] - c
    q = w * d * d
    col = lax.broadcasted_iota(jnp.int32, q.shape, 1)
    q_ref[...] = jnp.where(col == 0, 0.0, q)


def _compute_q(a, bt, bx, by, c, interpret=False):
    grid = (VPAD // ROWB,)
    return pl.pallas_call(
        _q_body,
        grid=grid,
        in_specs=[
            pl.BlockSpec((ROWB, DIM), lambda i: (i, 0)),
            pl.BlockSpec((DIM, VPAD), lambda i: (0, 0)),
            pl.BlockSpec((ROWB, 1), lambda i: (i, 0)),
            pl.BlockSpec((1, VPAD), lambda i: (0, 0)),
            pl.BlockSpec((ROWB, VPAD), lambda i: (i, 0)),
        ],
        out_specs=pl.BlockSpec((ROWB, VPAD), lambda i: (i, 0)),
        out_shape=jax.ShapeDtypeStruct((VPAD, VPAD), jnp.float32),
        interpret=interpret,
    )(a, bt, bx, by, c)


def _gather_sum_body(q_hbm, wdx_hbm, wdy_hbm, out_hbm,
                     xs_v, ys_v, rows_v, part_v, stage_v, acc_sh,
                     sem_r, sem_y):
    cid = lax.axis_index("c")
    sid = lax.axis_index("s")
    wid = sid * NC + cid
    base = wid * BPW

    pltpu.sync_copy(wdx_hbm.at[pl.ds(base, BPW)], xs_v)

    def chunk(ci, accs):
        b0 = base + ci * CH
        idxv = xs_v[pl.ds(ci * CH, CH)]
        cp_r = pltpu.make_async_copy(q_hbm.at[idxv], rows_v, sem_r)
        cp_r.start()
        cp_y = pltpu.make_async_copy(wdy_hbm.at[pl.ds(b0, CH)], ys_v, sem_y)
        cp_y.start()
        cp_y.wait()
        cp_r.wait()
        accs = list(accs)
        for j in range(CH):
            jidx = jnp.full((16,), j, jnp.int32)
            for v in range(NV):
                yv = ys_v[j, pl.ds(v * 16, 16)]
                k = (j * NV + v) % NACC
                accs[k] = accs[k] + plsc.load_gather(rows_v, [jidx, yv])
        return tuple(accs)

    zero = jnp.zeros((16,), jnp.float32)
    accs = lax.fori_loop(0, NCHUNK, chunk, (zero,) * NACC)
    total = accs[0]
    for k in range(1, NACC):
        total = total + accs[k]

    part_v[...] = total
    pltpu.sync_copy(part_v, acc_sh.at[sid])
    plsc.subcore_barrier()

    @pl.when(sid == 0)
    def _():
        pltpu.sync_copy(acc_sh, stage_v)
        t = jnp.zeros((16,), jnp.float32)
        for w in range(NS):
            t = t + stage_v[w]
        part_v[...] = t
        pltpu.sync_copy(part_v, out_hbm.at[cid])


def _gather_sum(q, wdx, wdy):
    mesh = plsc.VectorSubcoreMesh(core_axis_name="c", subcore_axis_name="s",
                                  num_cores=NC, num_subcores=NS)
    f = pl.kernel(
        _gather_sum_body,
        out_type=jax.ShapeDtypeStruct((NC, 16), jnp.float32),
        mesh=mesh,
        scratch_types=[
            pltpu.VMEM((BPW,), jnp.int32),
            pltpu.VMEM((CH, LPAD), jnp.int32),
            pltpu.VMEM((CH, VPAD), jnp.float32),
            pltpu.VMEM((16,), jnp.float32),
            pltpu.VMEM((NS, 16), jnp.float32),
            pltpu.VMEM_SHARED((NS, 16), jnp.float32),
            pltpu.SemaphoreType.DMA,
            pltpu.SemaphoreType.DMA,
        ],
        compiler_params=pltpu.CompilerParams(use_tc_tiling_on_sc=False,
                                             needs_layout_passes=False),
    )
    return f(q, wdx, wdy)


def kernel(wd_x, wd_y, in_emb, out_emb, in_bias, out_bias, co_matrix):
    a = jnp.pad(in_emb, ((0, VPAD - VOCAB), (0, 0)))
    bt = jnp.pad(out_emb, ((0, VPAD - VOCAB), (0, 0))).T
    bx = jnp.pad(in_bias, (0, VPAD - VOCAB)).reshape(VPAD, 1)
    by = jnp.pad(out_bias, (0, VPAD - VOCAB)).reshape(1, VPAD)
    c = jnp.pad(co_matrix, ((0, VPAD - VOCAB), (0, VPAD - VOCAB)))
    q = _compute_q(a, bt, bx, by, c)

    wdx = wd_x.astype(jnp.int32)
    wdy = jnp.pad(wd_y.astype(jnp.int32), ((0, 0), (0, LPAD - L)))
    out = _gather_sum(q, wdx, wdy)
    return jnp.sum(out)


# speedup vs baseline: 695.3225x; 1.2222x over previous
import functools

import jax
import jax.numpy as jnp
from jax import lax
from jax.experimental import pallas as pl
from jax.experimental.pallas import tpu as pltpu
from jax.experimental.pallas import tpu_sc as plsc

VOCAB = 1000
DIM = 64
B = 16384
L = 200
VPAD = 1024
ROWB = 128

NC = 2
NS = 16
NW = NC * NS
BPW = B // NW
CH = 16
NCHUNK = BPW // CH
NV = (L + 15) // 16
NACC = 8


def _q_body(a_ref, bt_ref, bx_ref, by_ref, c_ref, q_ref):
    m = lax.dot_general(a_ref[...], bt_ref[...], (((1,), (0,)), ((), ())),
                        preferred_element_type=jnp.float32)
    c = c_ref[...]
    w = jnp.where(c < 100.0, jnp.exp(0.75 * jnp.log(c * 0.01)), 1.0)
    d = m + bx_ref[...] + by_ref[...] - c
    q = w * d * d
    col = lax.broadcasted_iota(jnp.int32, q.shape, 1)
    q_ref[...] = jnp.where(col == 0, 0.0, q)


def _compute_q(a, bt, bx, by, c, interpret=False):
    grid = (VPAD // ROWB,)
    return pl.pallas_call(
        _q_body,
        grid=grid,
        in_specs=[
            pl.BlockSpec((ROWB, DIM), lambda i: (i, 0)),
            pl.BlockSpec((DIM, VPAD), lambda i: (0, 0)),
            pl.BlockSpec((ROWB, 1), lambda i: (i, 0)),
            pl.BlockSpec((1, VPAD), lambda i: (0, 0)),
            pl.BlockSpec((ROWB, VPAD), lambda i: (i, 0)),
        ],
        out_specs=pl.BlockSpec((ROWB, VPAD), lambda i: (i, 0)),
        out_shape=jax.ShapeDtypeStruct((VPAD, VPAD), jnp.float32),
        interpret=interpret,
    )(a, bt, bx, by, c)


def _gather_sum_body(q_hbm, wdx_hbm, wdy_hbm, out_hbm,
                     xs_v, ys0, ys1, rows0, rows1, part_v, stage_v, acc_sh,
                     sem_r0, sem_y0, sem_r1, sem_y1):
    cid = lax.axis_index("c")
    sid = lax.axis_index("s")
    wid = sid * NC + cid
    base = wid * BPW
    chl = CH * L

    pltpu.sync_copy(wdx_hbm.at[pl.ds(base, BPW)], xs_v)

    def mk(ci, rows_v, ys_v, sr, sy):
        idxv = xs_v[pl.ds(ci * CH, CH)]
        cp_r = pltpu.make_async_copy(q_hbm.at[idxv], rows_v, sr)
        cp_y = pltpu.make_async_copy(
            wdy_hbm.at[pl.ds((base + ci * CH) * L, chl)],
            ys_v.at[pl.ds(0, chl)], sy)
        return cp_r, cp_y

    def start(ci, rows_v, ys_v, sr, sy):
        cp_r, cp_y = mk(ci, rows_v, ys_v, sr, sy)
        cp_r.start()
        cp_y.start()

    def wait(ci, rows_v, ys_v, sr, sy):
        cp_r, cp_y = mk(ci, rows_v, ys_v, sr, sy)
        cp_y.wait()
        cp_r.wait()

    tail_mask = lax.iota(jnp.int32, 16) < (L - (NV - 1) * 16)

    def compute(rows_v, ys_v, accs):
        accs = list(accs)
        for j in range(CH):
            jidx = jnp.full((16,), j, jnp.int32)
            for v in range(NV):
                yv = ys_v[pl.ds(j * L + v * 16, 16)]
                if v == NV - 1:
                    yv = jnp.where(tail_mask, yv, 0)
                k = (j * NV + v) % NACC
                accs[k] = accs[k] + plsc.load_gather(rows_v, [jidx, yv])
        return tuple(accs)

    start(0, rows0, ys0, sem_r0, sem_y0)

    def pair(pi, accs):
        ci0 = 2 * pi
        wait(ci0, rows0, ys0, sem_r0, sem_y0)
        start(ci0 + 1, rows1, ys1, sem_r1, sem_y1)
        accs = compute(rows0, ys0, accs)
        wait(ci0 + 1, rows1, ys1, sem_r1, sem_y1)

        @pl.when(pi < NCHUNK // 2 - 1)
        def _():
            start(ci0 + 2, rows0, ys0, sem_r0, sem_y0)

        return compute(rows1, ys1, accs)

    zero = jnp.zeros((16,), jnp.float32)
    accs = lax.fori_loop(0, NCHUNK // 2, pair, (zero,) * NACC)
    total = accs[0]
    for k in range(1, NACC):
        total = total + accs[k]

    part_v[...] = total
    pltpu.sync_copy(part_v, acc_sh.at[sid])
    plsc.subcore_barrier()

    @pl.when(sid == 0)
    def _():
        pltpu.sync_copy(acc_sh, stage_v)
        t = jnp.zeros((16,), jnp.float32)
        for w in range(NS):
            t = t + stage_v[w]
        part_v[...] = t
        pltpu.sync_copy(part_v, out_hbm.at[cid])


def _gather_sum(q, wdx, wdy):
    mesh = plsc.VectorSubcoreMesh(core_axis_name="c", subcore_axis_name="s",
                                  num_cores=NC, num_subcores=NS)
    f = pl.kernel(
        _gather_sum_body,
        out_type=jax.ShapeDtypeStruct((NC, 16), jnp.float32),
        mesh=mesh,
        scratch_types=[
            pltpu.VMEM((BPW,), jnp.int32),
            pltpu.VMEM((CH * L + 16,), jnp.int32),
            pltpu.VMEM((CH * L + 16,), jnp.int32),
            pltpu.VMEM((CH, VPAD), jnp.float32),
            pltpu.VMEM((CH, VPAD), jnp.float32),
            pltpu.VMEM((16,), jnp.float32),
            pltpu.VMEM((NS, 16), jnp.float32),
            pltpu.VMEM_SHARED((NS, 16), jnp.float32),
            pltpu.SemaphoreType.DMA,
            pltpu.SemaphoreType.DMA,
            pltpu.SemaphoreType.DMA,
            pltpu.SemaphoreType.DMA,
        ],
        compiler_params=pltpu.CompilerParams(use_tc_tiling_on_sc=False,
                                             needs_layout_passes=False),
    )
    return f(q, wdx, wdy)


def kernel(wd_x, wd_y, in_emb, out_emb, in_bias, out_bias, co_matrix):
    a = jnp.pad(in_emb, ((0, VPAD - VOCAB), (0, 0)))
    bt = jnp.pad(out_emb, ((0, VPAD - VOCAB), (0, 0))).T
    bx = jnp.pad(in_bias, (0, VPAD - VOCAB)).reshape(VPAD, 1)
    by = jnp.pad(out_bias, (0, VPAD - VOCAB)).reshape(1, VPAD)
    c = jnp.pad(co_matrix, ((0, VPAD - VOCAB), (0, VPAD - VOCAB)))
    q = _compute_q(a, bt, bx, by, c)

    wdx = wd_x.astype(jnp.int32)
    wdy = wd_y.astype(jnp.int32).reshape(-1)
    out = _gather_sum(q, wdx, wdy)
    return jnp.sum(out)


# speedup vs baseline: 726.6273x; 1.0450x over previous
import functools

import jax
import jax.numpy as jnp
from jax import lax
from jax.experimental import pallas as pl
from jax.experimental.pallas import tpu as pltpu
from jax.experimental.pallas import tpu_sc as plsc

VOCAB = 1000
DIM = 64
B = 16384
L = 200
VPAD = 1024
ROWB = 128

NC = 2
NS = 16
NW = NC * NS
BPW = B // NW
CH = 16
NCHUNK = BPW // CH
NV = (L + 15) // 16
NACC = 8


def _q_body(a_ref, bt_ref, bx_ref, by_ref, c_ref, q_ref):
    m = lax.dot_general(a_ref[...], bt_ref[...], (((1,), (0,)), ((), ())),
                        preferred_element_type=jnp.float32)
    c = c_ref[...]
    w = jnp.where(c < 100.0, jnp.exp(0.75 * jnp.log(c * 0.01)), 1.0)
    d = m + bx_ref[...] + by_ref[...] - c
    q = w * d * d
    col = lax.broadcasted_iota(jnp.int32, q.shape, 1)
    q_ref[...] = jnp.where(col == 0, 0.0, q)


def _compute_q(a, bt, bx, by, c, interpret=False):
    grid = (VPAD // ROWB,)
    return pl.pallas_call(
        _q_body,
        grid=grid,
        in_specs=[
            pl.BlockSpec((ROWB, DIM), lambda i: (i, 0)),
            pl.BlockSpec((DIM, VPAD), lambda i: (0, 0)),
            pl.BlockSpec((ROWB, 1), lambda i: (i, 0)),
            pl.BlockSpec((1, VPAD), lambda i: (0, 0)),
            pl.BlockSpec((ROWB, VPAD), lambda i: (i, 0)),
        ],
        out_specs=pl.BlockSpec((ROWB, VPAD), lambda i: (i, 0)),
        out_shape=jax.ShapeDtypeStruct((VPAD, VPAD), jnp.float32),
        interpret=interpret,
    )(a, bt, bx, by, c)


def _gather_sum_body(q_hbm, wdx_hbm, wdy_hbm, out_hbm,
                     xs_v, ys0, ys1, rows0, rows1, part_v, stage_v, acc_sh,
                     sem_r0, sem_y0, sem_r1, sem_y1):
    cid = lax.axis_index("c")
    sid = lax.axis_index("s")
    wid = sid * NC + cid
    base = wid * BPW
    chl = CH * L

    pltpu.sync_copy(wdx_hbm.at[pl.ds(base, BPW)], xs_v)

    def mk(ci, rows_v, ys_v, sr, sy):
        idxv = xs_v[pl.ds(ci * CH, CH)]
        cp_r = pltpu.make_async_copy(q_hbm.at[idxv], rows_v, sr)
        cp_y = pltpu.make_async_copy(
            wdy_hbm.at[pl.ds((base + ci * CH) * L, chl)],
            ys_v.at[pl.ds(0, chl)], sy)
        return cp_r, cp_y

    def start(ci, rows_v, ys_v, sr, sy):
        cp_r, cp_y = mk(ci, rows_v, ys_v, sr, sy)
        cp_r.start()
        cp_y.start()

    def wait(ci, rows_v, ys_v, sr, sy):
        cp_r, cp_y = mk(ci, rows_v, ys_v, sr, sy)
        cp_y.wait()
        cp_r.wait()

    tail_mask = lax.iota(jnp.int32, 16) < (L - (NV - 1) * 16)

    def compute(rows_v, ys_v, accs):
        accs = list(accs)
        for j in range(CH):
            jidx = jnp.full((16,), j, jnp.int32)
            for v in range(NV):
                yv = ys_v[pl.ds(j * L + v * 16, 16)]
                if v == NV - 1:
                    yv = jnp.where(tail_mask, yv, 0)
                k = (j * NV + v) % NACC
                accs[k] = accs[k] + plsc.load_gather(rows_v, [jidx, yv])
        return tuple(accs)

    start(0, rows0, ys0, sem_r0, sem_y0)

    def pair(pi, accs):
        ci0 = 2 * pi
        wait(ci0, rows0, ys0, sem_r0, sem_y0)
        start(ci0 + 1, rows1, ys1, sem_r1, sem_y1)
        accs = compute(rows0, ys0, accs)
        wait(ci0 + 1, rows1, ys1, sem_r1, sem_y1)

        @pl.when(pi < NCHUNK // 2 - 1)
        def _():
            start(ci0 + 2, rows0, ys0, sem_r0, sem_y0)

        return compute(rows1, ys1, accs)

    zero = jnp.zeros((16,), jnp.float32)
    accs = lax.fori_loop(0, NCHUNK // 2, pair, (zero,) * NACC)
    total = accs[0]
    for k in range(1, NACC):
        total = total + accs[k]

    part_v[...] = total
    pltpu.sync_copy(part_v, acc_sh.at[sid])
    plsc.subcore_barrier()

    @pl.when(sid == 0)
    def _():
        pltpu.sync_copy(acc_sh, stage_v)
        t = jnp.zeros((16,), jnp.float32)
        for w in range(NS):
            t = t + stage_v[w]
        part_v[...] = t
        pltpu.sync_copy(part_v, out_hbm.at[cid])


def _gather_sum(q, wdx, wdy):
    mesh = plsc.VectorSubcoreMesh(core_axis_name="c", subcore_axis_name="s",
                                  num_cores=NC, num_subcores=NS)
    f = pl.kernel(
        _gather_sum_body,
        out_type=jax.ShapeDtypeStruct((NC, 16), jnp.float32),
        mesh=mesh,
        scratch_types=[
            pltpu.VMEM((BPW,), jnp.int32),
            pltpu.VMEM((CH * L + 16,), jnp.int32),
            pltpu.VMEM((CH * L + 16,), jnp.int32),
            pltpu.VMEM((CH, VPAD), jnp.float32),
            pltpu.VMEM((CH, VPAD), jnp.float32),
            pltpu.VMEM((16,), jnp.float32),
            pltpu.VMEM((NS, 16), jnp.float32),
            pltpu.VMEM_SHARED((NS, 16), jnp.float32),
            pltpu.SemaphoreType.DMA,
            pltpu.SemaphoreType.DMA,
            pltpu.SemaphoreType.DMA,
            pltpu.SemaphoreType.DMA,
        ],
        compiler_params=pltpu.CompilerParams(use_tc_tiling_on_sc=False,
                                             needs_layout_passes=False),
    )
    return f(q, wdx, wdy)


def kernel(wd_x, wd_y, in_emb, out_emb, in_bias, out_bias, co_matrix):
    bt = out_emb.T
    bx = in_bias.reshape(VOCAB, 1)
    by = out_bias.reshape(1, VOCAB)
    q = _compute_q(in_emb, bt, bx, by, co_matrix)

    wdx = wd_x.astype(jnp.int32)
    wdy = wd_y.astype(jnp.int32).reshape(-1)
    out = _gather_sum(q, wdx, wdy)
    return jnp.sum(out)
